# baseline (speedup 1.0000x reference)
import functools

import jax
import jax.numpy as jnp
from jax import lax
from jax.experimental import pallas as pl
from jax.experimental.pallas import tpu as pltpu
from jax.experimental.pallas import tpu_sc as plsc

BATCH = 16384
EMBED_DIM = 64


@functools.cache
def _build_lookup():
    info = plsc.get_sparse_core_info()
    num_workers = info.num_cores * info.num_subcores
    b_per_w = BATCH // num_workers
    mesh = plsc.VectorSubcoreMesh(core_axis_name="c", subcore_axis_name="s")

    @functools.partial(
        pl.kernel,
        mesh=mesh,
        out_type=(
            jax.ShapeDtypeStruct((BATCH, EMBED_DIM), jnp.float32),
            jax.ShapeDtypeStruct((BATCH, EMBED_DIM), jnp.float32),
        ),
        scratch_types=[
            pltpu.VMEM((b_per_w,), jnp.int32),
            pltpu.VMEM((b_per_w,), jnp.int32),
            pltpu.SemaphoreType.DMA,
            pltpu.SemaphoreType.DMA,
        ],
    )
    def lookup(uids_hbm, iids_hbm, utab_hbm, itab_hbm, uout_hbm, iout_hbm,
               uidx_v, iidx_v, usem, isem):
        wid = lax.axis_index("s") * info.num_cores + lax.axis_index("c")
        base = wid * b_per_w
        pltpu.sync_copy(uids_hbm.at[pl.ds(base, b_per_w)], uidx_v)
        pltpu.sync_copy(iids_hbm.at[pl.ds(base, b_per_w)], iidx_v)

        def issue(g, _):
            uvec = uidx_v[pl.ds(g * 16, 16)]
            ivec = iidx_v[pl.ds(g * 16, 16)]
            for j in range(16):
                urow = uvec[j]
                irow = ivec[j]
                pltpu.async_copy(utab_hbm.at[pl.ds(urow, 1)],
                                 uout_hbm.at[pl.ds(base + g * 16 + j, 1)], usem)
                pltpu.async_copy(itab_hbm.at[pl.ds(irow, 1)],
                                 iout_hbm.at[pl.ds(base + g * 16 + j, 1)], isem)
            return _

        lax.fori_loop(0, b_per_w // 16, issue, None)

        def drain(i, _):
            pltpu.make_async_copy(utab_hbm.at[pl.ds(0, 1)],
                                  uout_hbm.at[pl.ds(base + i, 1)], usem).wait()
            pltpu.make_async_copy(itab_hbm.at[pl.ds(0, 1)],
                                  iout_hbm.at[pl.ds(base + i, 1)], isem).wait()
            return _

        lax.fori_loop(0, b_per_w, drain, None)

    return lookup


def kernel(user_ids, item_ids, user_table, item_table):
    lookup = _build_lookup()
    return lookup(user_ids, item_ids, user_table, item_table)


# speedup vs baseline: 1.6826x; 1.6826x over previous
import functools

import jax
import jax.numpy as jnp
from jax import lax
from jax.experimental import pallas as pl
from jax.experimental.pallas import tpu as pltpu
from jax.experimental.pallas import tpu_sc as plsc

BATCH = 16384
EMBED_DIM = 64


@functools.cache
def _build_lookup():
    info = plsc.get_sparse_core_info()
    num_workers = info.num_cores * info.num_subcores
    b_per_w = BATCH // num_workers
    mesh = plsc.VectorSubcoreMesh(core_axis_name="c", subcore_axis_name="s")

    @functools.partial(
        pl.kernel,
        mesh=mesh,
        out_type=(
            jax.ShapeDtypeStruct((BATCH, EMBED_DIM), jnp.float32),
            jax.ShapeDtypeStruct((BATCH, EMBED_DIM), jnp.float32),
        ),
        scratch_types=[
            pltpu.VMEM((b_per_w,), jnp.int32),
            pltpu.VMEM((b_per_w,), jnp.int32),
            pltpu.VMEM((b_per_w // 2, EMBED_DIM), jnp.float32),
            pltpu.VMEM((b_per_w // 2, EMBED_DIM), jnp.float32),
            pltpu.SemaphoreType.DMA,
            pltpu.SemaphoreType.DMA,
        ],
    )
    def lookup(uids_hbm, iids_hbm, utab_hbm, itab_hbm, uout_hbm, iout_hbm,
               uidx_v, iidx_v, urows_v, irows_v, usem, isem):
        wid = lax.axis_index("s") * info.num_cores + lax.axis_index("c")
        base = wid * b_per_w
        pltpu.sync_copy(uids_hbm.at[pl.ds(base, b_per_w)], uidx_v)
        pltpu.sync_copy(iids_hbm.at[pl.ds(base, b_per_w)], iidx_v)

        half = b_per_w // 2

        def issue(g, _):
            uvec = uidx_v[pl.ds(g * 16, 16)]
            ivec = iidx_v[pl.ds(g * 16, 16)]
            for j in range(16):
                row = (g * 16 + j) % half
                pltpu.async_copy(utab_hbm.at[pl.ds(uvec[j], 1)],
                                 urows_v.at[pl.ds(row, 1)], usem)
                pltpu.async_copy(itab_hbm.at[pl.ds(ivec[j], 1)],
                                 irows_v.at[pl.ds(row, 1)], isem)
            return _

        def drain(i, _):
            pltpu.make_async_copy(utab_hbm.at[pl.ds(0, 1)],
                                  urows_v.at[pl.ds(i, 1)], usem).wait()
            pltpu.make_async_copy(itab_hbm.at[pl.ds(0, 1)],
                                  irows_v.at[pl.ds(i, 1)], isem).wait()
            return _

        for p in range(2):
            lax.fori_loop(p * (half // 16), (p + 1) * (half // 16), issue, None)
            lax.fori_loop(0, half, drain, None)
            pltpu.sync_copy(urows_v, uout_hbm.at[pl.ds(base + p * half, half)])
            pltpu.sync_copy(irows_v, iout_hbm.at[pl.ds(base + p * half, half)])

    return lookup


def kernel(user_ids, item_ids, user_table, item_table):
    lookup = _build_lookup()
    return lookup(user_ids, item_ids, user_table, item_table)


# speedup vs baseline: 3.8989x; 2.3172x over previous
import functools

import jax
import jax.numpy as jnp
from jax import lax
from jax.experimental import pallas as pl
from jax.experimental.pallas import tpu as pltpu
from jax.experimental.pallas import tpu_sc as plsc

BATCH = 16384
EMBED_DIM = 64
NUM_IDS = 1000000
IDS_PER_W = 31232
WIN = 512
TAIL = NUM_IDS - 32 * IDS_PER_W
SENTINEL = 1 << 30
RING = 64


@functools.cache
def _build_lookup():
    info = plsc.get_sparse_core_info()
    num_workers = info.num_cores * info.num_subcores
    mesh = plsc.VectorSubcoreMesh(core_axis_name="c", subcore_axis_name="s")

    @functools.partial(
        pl.kernel,
        mesh=mesh,
        out_type=(
            jax.ShapeDtypeStruct((BATCH, EMBED_DIM), jnp.float32),
            jax.ShapeDtypeStruct((BATCH, EMBED_DIM), jnp.float32),
        ),
        scratch_types=[
            pltpu.VMEM((BATCH + 64,), jnp.int32),
            pltpu.VMEM((BATCH + 64,), jnp.int32),
            pltpu.VMEM((64,), jnp.int32),
            pltpu.VMEM((EMBED_DIM, WIN), jnp.float32),
            pltpu.VMEM((EMBED_DIM, WIN), jnp.float32),
            pltpu.VMEM((EMBED_DIM, 128), jnp.float32),
            pltpu.VMEM((RING, EMBED_DIM), jnp.float32),
            pltpu.SemaphoreType.DMA,
            pltpu.SemaphoreType.DMA,
            pltpu.SemaphoreType.DMA,
        ],
        compiler_params=pltpu.CompilerParams(needs_layout_passes=False),
    )
    def lookup(usid_hbm, uord_hbm, ustarts_hbm, isid_hbm, iord_hbm,
               istarts_hbm, utab_hbm, itab_hbm, uout_hbm, iout_hbm,
               sid_v, ord_v, starts_v, buf0, buf1, buf2, ring,
               sem0, sem1, wsem):
        wid = lax.axis_index("s") * info.num_cores + lax.axis_index("c")
        base_id = wid * IDS_PER_W
        lanes = lax.iota(jnp.int32, 16)

        def run_table(tab, sid_hbm, ord_hbm, starts_hbm, out_hbm):
            pltpu.sync_copy(starts_hbm, starts_v)
            pltpu.sync_copy(sid_hbm, sid_v)
            pltpu.sync_copy(ord_hbm, ord_v)
            p0 = plsc.load_gather(
                starts_v, [jnp.broadcast_to(wid, (16,))])[0]

            def drain_one(i, x):
                pltpu.make_async_copy(ring.at[pl.ds(0, 1)],
                                      out_hbm.at[pl.ds(0, 1)], wsem).wait()
                return x

            def proc_window(buf, win_start, width, carry):
                def cond(c):
                    return c[3]

                def body(c):
                    p, fired, prev, _ = c
                    svec = sid_v[pl.ds(p, 16)]
                    jvec = ord_v[pl.ds(p, 16)]
                    m = (svec >= win_start) & (svec < win_start + width)
                    cnt = plsc.all_reduce_population_count(m)[0]
                    for l in range(16):
                        @pl.when(l < cnt)
                        def _():
                            col = jnp.broadcast_to(svec[l] - win_start, (16,))
                            slot = lax.rem(fired + l, RING)
                            srow = jnp.broadcast_to(slot, (16,))
                            for mm in range(4):
                                rows = lanes + 16 * mm
                                g = plsc.load_gather(buf, [rows, col])
                                plsc.store_scatter(ring, [srow, rows], g)
                            pltpu.async_copy(
                                ring.at[pl.ds(slot, 1)],
                                out_hbm.at[pl.ds(jvec[l], 1)], wsem)
                    lax.fori_loop(0, prev, drain_one, 0)
                    return (p + cnt, fired + cnt, cnt, cnt == 16)

                return lax.while_loop(cond, body, carry[:3] + (True,))[:3]

            pltpu.async_copy(tab.at[:, pl.ds(base_id, WIN)], buf0, sem0)
            carry = (p0, jnp.int32(0), jnp.int32(0))

            def t_body(t, carry):
                w1 = base_id + (2 * t + 1) * WIN
                pltpu.async_copy(tab.at[:, pl.ds(w1, WIN)], buf1, sem1)
                pltpu.make_async_copy(tab.at[:, pl.ds(0, WIN)], buf0,
                                      sem0).wait()
                carry = proc_window(buf0, base_id + 2 * t * WIN, WIN, carry)

                @pl.when(t < 30)
                def _():
                    pltpu.async_copy(
                        tab.at[:, pl.ds(base_id + (2 * t + 2) * WIN, WIN)],
                        buf0, sem0)
                pltpu.make_async_copy(tab.at[:, pl.ds(0, WIN)], buf1,
                                      sem1).wait()
                carry = proc_window(buf1, w1, WIN, carry)
                return carry

            carry = lax.fori_loop(0, 31, t_body, carry)
            tail_start = base_id + 62 * WIN
            pltpu.sync_copy(tab.at[:, pl.ds(tail_start, 128)], buf2)
            carry = proc_window(buf2, tail_start, 128, carry)
            lax.fori_loop(0, carry[2], drain_one, 0)

        run_table(utab_hbm, usid_hbm, uord_hbm, ustarts_hbm, uout_hbm)
        run_table(itab_hbm, isid_hbm, iord_hbm, istarts_hbm, iout_hbm)

    return lookup


def _prep(ids):
    ids = ids.astype(jnp.int32)
    sid, order = lax.sort(
        (ids, jnp.arange(BATCH, dtype=jnp.int32)), num_keys=1)
    bounds = jnp.arange(33, dtype=jnp.int32) * IDS_PER_W
    starts = jnp.searchsorted(sid, bounds).astype(jnp.int32)
    starts = jnp.pad(starts, (0, 31))
    pad = jnp.full((64,), SENTINEL, jnp.int32)
    return (jnp.concatenate([sid, pad]),
            jnp.concatenate([order, jnp.zeros((64,), jnp.int32)]),
            starts)


def kernel(user_ids, item_ids, user_table, item_table):
    lookup = _build_lookup()
    usid, uord, ustarts = _prep(user_ids)
    isid, iord, istarts = _prep(item_ids)
    return lookup(usid, uord, ustarts, isid, iord, istarts,
                  user_table.T, item_table.T)


# speedup vs baseline: 3.9939x; 1.0244x over previous
import functools

import jax
import jax.numpy as jnp
from jax import lax
from jax.experimental import pallas as pl
from jax.experimental.pallas import tpu as pltpu
from jax.experimental.pallas import tpu_sc as plsc

BATCH = 16384
EMBED_DIM = 64
NUM_IDS = 1000000
IDS_PER_W = 31232
WIN = 512
TAIL = NUM_IDS - 32 * IDS_PER_W
SENTINEL = 1 << 30
RING = 64


@functools.cache
def _build_lookup():
    info = plsc.get_sparse_core_info()
    num_workers = info.num_cores * info.num_subcores
    mesh = plsc.VectorSubcoreMesh(core_axis_name="c", subcore_axis_name="s")

    @functools.partial(
        pl.kernel,
        mesh=mesh,
        out_type=(
            jax.ShapeDtypeStruct((BATCH, EMBED_DIM), jnp.float32),
            jax.ShapeDtypeStruct((BATCH, EMBED_DIM), jnp.float32),
        ),
        scratch_types=[
            pltpu.VMEM((BATCH + 64,), jnp.int32),
            pltpu.VMEM((BATCH + 64,), jnp.int32),
            pltpu.VMEM((64,), jnp.int32),
            pltpu.VMEM((EMBED_DIM, WIN), jnp.float32),
            pltpu.VMEM((EMBED_DIM, WIN), jnp.float32),
            pltpu.VMEM((EMBED_DIM, 128), jnp.float32),
            pltpu.VMEM((RING, EMBED_DIM), jnp.float32),
            pltpu.SemaphoreType.DMA,
            pltpu.SemaphoreType.DMA,
            pltpu.SemaphoreType.DMA,
            pltpu.SemaphoreType.DMA,
            pltpu.SemaphoreType.DMA,
        ],
        compiler_params=pltpu.CompilerParams(needs_layout_passes=False),
    )
    def lookup(usid_hbm, uord_hbm, ustarts_hbm, isid_hbm, iord_hbm,
               istarts_hbm, utab_hbm, itab_hbm, uout_hbm, iout_hbm,
               sid_v, ord_v, starts_v, buf0, buf1, buf2, ring,
               sem0, sem0b, sem1, sem1b, wsem):
        wid = lax.axis_index("s") * info.num_cores + lax.axis_index("c")
        base_id = wid * IDS_PER_W
        lanes = lax.iota(jnp.int32, 16)

        def win_fetch(tab, start, buf, sa, sb):
            pltpu.async_copy(tab.at[pl.ds(0, 32), pl.ds(start, WIN)],
                             buf.at[pl.ds(0, 32)], sa)
            pltpu.async_copy(tab.at[pl.ds(32, 32), pl.ds(start, WIN)],
                             buf.at[pl.ds(32, 32)], sb)

        def win_wait(tab, buf, sa, sb):
            pltpu.make_async_copy(tab.at[pl.ds(0, 32), pl.ds(0, WIN)],
                                  buf.at[pl.ds(0, 32)], sa).wait()
            pltpu.make_async_copy(tab.at[pl.ds(32, 32), pl.ds(0, WIN)],
                                  buf.at[pl.ds(32, 32)], sb).wait()

        def run_table(tab, sid_hbm, ord_hbm, starts_hbm, out_hbm):
            pltpu.sync_copy(starts_hbm, starts_v)
            pltpu.sync_copy(sid_hbm, sid_v)
            pltpu.sync_copy(ord_hbm, ord_v)
            p0 = plsc.load_gather(
                starts_v, [jnp.broadcast_to(wid, (16,))])[0]

            def drain_one(i, x):
                pltpu.make_async_copy(ring.at[pl.ds(0, 1)],
                                      out_hbm.at[pl.ds(0, 1)], wsem).wait()
                return x

            def proc_window(buf, win_start, width, carry):
                def cond(c):
                    return c[3]

                def body(c):
                    p, fired, prev, _ = c
                    svec = sid_v[pl.ds(p, 16)]
                    jvec = ord_v[pl.ds(p, 16)]
                    m = (svec >= win_start) & (svec < win_start + width)
                    cnt = plsc.all_reduce_population_count(m)[0]
                    for l in range(16):
                        @pl.when(l < cnt)
                        def _():
                            col = jnp.broadcast_to(svec[l] - win_start, (16,))
                            slot = lax.rem(fired + l, RING)
                            srow = jnp.broadcast_to(slot, (16,))
                            for mm in range(4):
                                rows = lanes + 16 * mm
                                g = plsc.load_gather(buf, [rows, col])
                                plsc.store_scatter(ring, [srow, rows], g)
                            pltpu.async_copy(
                                ring.at[pl.ds(slot, 1)],
                                out_hbm.at[pl.ds(jvec[l], 1)], wsem)
                    lax.fori_loop(0, prev, drain_one, 0)
                    return (p + cnt, fired + cnt, cnt, cnt == 16)

                return lax.while_loop(cond, body, carry[:3] + (True,))[:3]

            win_fetch(tab, base_id, buf0, sem0, sem0b)
            carry = (p0, jnp.int32(0), jnp.int32(0))

            def t_body(t, carry):
                w1 = base_id + (2 * t + 1) * WIN
                win_fetch(tab, w1, buf1, sem1, sem1b)
                win_wait(tab, buf0, sem0, sem0b)
                carry = proc_window(buf0, base_id + 2 * t * WIN, WIN, carry)

                @pl.when(t < 30)
                def _():
                    win_fetch(tab, base_id + (2 * t + 2) * WIN, buf0,
                              sem0, sem0b)
                win_wait(tab, buf1, sem1, sem1b)
                carry = proc_window(buf1, w1, WIN, carry)
                return carry

            carry = lax.fori_loop(0, 31, t_body, carry)
            tail_start = base_id + 62 * WIN
            pltpu.sync_copy(tab.at[:, pl.ds(tail_start, 128)], buf2)
            carry = proc_window(buf2, tail_start, 128, carry)
            lax.fori_loop(0, carry[2], drain_one, 0)

        run_table(utab_hbm, usid_hbm, uord_hbm, ustarts_hbm, uout_hbm)
        run_table(itab_hbm, isid_hbm, iord_hbm, istarts_hbm, iout_hbm)

    return lookup


def _prep(ids):
    ids = ids.astype(jnp.int32)
    sid, order = lax.sort(
        (ids, jnp.arange(BATCH, dtype=jnp.int32)), num_keys=1)
    bounds = jnp.arange(33, dtype=jnp.int32) * IDS_PER_W
    starts = jnp.searchsorted(sid, bounds).astype(jnp.int32)
    starts = jnp.pad(starts, (0, 31))
    pad = jnp.full((64,), SENTINEL, jnp.int32)
    return (jnp.concatenate([sid, pad]),
            jnp.concatenate([order, jnp.zeros((64,), jnp.int32)]),
            starts)


def kernel(user_ids, item_ids, user_table, item_table):
    lookup = _build_lookup()
    usid, uord, ustarts = _prep(user_ids)
    isid, iord, istarts = _prep(item_ids)
    return lookup(usid, uord, ustarts, isid, iord, istarts,
                  user_table.T, item_table.T)
